# baseline (speedup 1.0000x reference)
import functools
import jax
import jax.numpy as jnp
from jax import lax
from jax.experimental import pallas as pl
from jax.experimental.pallas import tpu as pltpu
from jax.experimental.pallas import tpu_sc as plsc

_N = 10000
_E = 80000
_A = 20000
_N2 = 10000
_E2 = 80000
_G = 512
_NUM_FEAT = 64
_B_IN = 16
_DIM = 64
_NUM_I2 = 32
_W_REG = 1e-6
_D_REG = 1e-5

_NC = 2
_NS = 16
_NW = _NC * _NS
_CH = 128

_EPAD = 81920
_NPAD = 12288
_APAD = 20480


def _group_size(ncw):
  for k in (5, 4, 3, 2):
    if ncw % k == 0:
      return k
  return 1


@functools.lru_cache(maxsize=None)
def _make_sc_gather(f, mpad, vrows):
  ncw = mpad // (_NW * _CH)
  kk = 10 if ncw % 10 == 0 else _group_size(ncw)
  mesh = plsc.VectorSubcoreMesh(core_axis_name="c", subcore_axis_name="s")

  def body(table_hbm, idx_hbm, out_hbm, idx_v, buf_v, table_spm, sem):
    c = lax.axis_index("c")
    s = lax.axis_index("s")
    wid = s * _NC + c
    trows = vrows // _NS
    pltpu.sync_copy(table_hbm.at[pl.ds(s * trows, trows)],
                    table_spm.at[pl.ds(s * trows, trows)])
    pltpu.sync_copy(idx_hbm.at[wid], idx_v)
    plsc.subcore_barrier()
    base = wid * (ncw * _CH)

    def group(g, carry):
      j0 = g * kk
      handles = [
          pltpu.async_copy(table_spm.at[idx_v.at[j0 + b]],
                           buf_v.at[pl.ds(b * _CH, _CH)], sem)
          for b in range(kk)
      ]
      for b, h in enumerate(handles):
        h.wait()
        pltpu.sync_copy(buf_v.at[pl.ds(b * _CH, _CH)],
                        out_hbm.at[pl.ds(base + (j0 + b) * _CH, _CH)])
      return carry

    lax.fori_loop(0, ncw // kk, group, 0)

  return pl.kernel(
      body,
      out_type=jax.ShapeDtypeStruct((mpad, f), jnp.float32),
      mesh=mesh,
      compiler_params=pltpu.CompilerParams(use_tc_tiling_on_sc=False),
      scratch_types=[
          pltpu.VMEM((ncw, _CH), jnp.int32),
          pltpu.VMEM((kk * _CH, f), jnp.float32),
          pltpu.VMEM_SHARED((vrows, f), jnp.float32),
          pltpu.SemaphoreType.DMA,
      ],
  )


@functools.lru_cache(maxsize=None)
def _make_sc_scatter_add(f, mpad, nout, indirect, vrows=0):
  ncw = mpad // (_NW * _CH)
  rows = nout // _NS
  tab_words = (vrows * f) if indirect else 0
  kk = _group_size(ncw)
  for cand in (10, 5):
    if cand <= kk or ncw % cand:
      continue
    spmem_words = (_NS * (2 * ncw * _CH + cand * _CH * f)
                   + (nout + 8) * f + tab_words)
    if spmem_words < 2_050_000:
      kk = cand
      break
  mesh = plsc.VectorSubcoreMesh(core_axis_name="c", subcore_axis_name="s")

  def run(vals_hbm, src_hbm, dst_hbm, zeros_hbm, out_hbm,
          src_v, dst_v, buf_v, accum, table_spm, sem):
    c = lax.axis_index("c")
    s = lax.axis_index("s")
    wid = s * _NC + c
    pltpu.sync_copy(zeros_hbm, accum.at[pl.ds(s * rows, rows)])
    pltpu.sync_copy(zeros_hbm.at[pl.ds(0, 8)], accum.at[pl.ds(nout, 8)])
    if indirect:
      trows = vrows // _NS
      pltpu.sync_copy(vals_hbm.at[pl.ds(s * trows, trows)],
                      table_spm.at[pl.ds(s * trows, trows)])
      pltpu.sync_copy(src_hbm.at[wid], src_v)
    plsc.subcore_barrier()
    pltpu.sync_copy(dst_hbm.at[wid], dst_v)
    base = wid * (ncw * _CH)

    def group(g, carry):
      j0 = g * kk
      if indirect:
        handles = [
            pltpu.async_copy(table_spm.at[src_v.at[j0 + b]],
                             buf_v.at[pl.ds(b * _CH, _CH)], sem)
            for b in range(kk)
        ]
      else:
        handles = [
            pltpu.async_copy(vals_hbm.at[pl.ds(base + (j0 + b) * _CH, _CH)],
                             buf_v.at[pl.ds(b * _CH, _CH)], sem)
            for b in range(kk)
        ]
      for b, h in enumerate(handles):
        h.wait()
        pltpu.sync_copy(buf_v.at[pl.ds(b * _CH, _CH)],
                        accum.at[dst_v.at[j0 + b]], add=True)
      return carry

    lax.fori_loop(0, ncw // kk, group, 0)
    plsc.subcore_barrier()
    pltpu.sync_copy(accum.at[pl.ds(s * rows, rows)],
                    out_hbm.at[c, pl.ds(s * rows, rows)])

  scratch = [
      pltpu.VMEM((ncw, _CH), jnp.int32),
      pltpu.VMEM((ncw, _CH), jnp.int32),
      pltpu.VMEM((kk * _CH, f), jnp.float32),
      pltpu.VMEM_SHARED((nout + 8, f), jnp.float32),
  ]
  if indirect:
    body = run
    scratch.append(pltpu.VMEM_SHARED((vrows, f), jnp.float32))
  else:
    def body(vals_hbm, dst_hbm, zeros_hbm, out_hbm,
             src_v, dst_v, buf_v, accum, sem):
      run(vals_hbm, None, dst_hbm, zeros_hbm, out_hbm,
          src_v, dst_v, buf_v, accum, None, sem)
  scratch.append(pltpu.SemaphoreType.DMA)

  return pl.kernel(
      body,
      out_type=jax.ShapeDtypeStruct((_NC, nout, f), jnp.float32),
      mesh=mesh,
      compiler_params=pltpu.CompilerParams(use_tc_tiling_on_sc=False),
      scratch_types=scratch,
  )


def _pad_idx(idx, mpad, fill, parts=1):
  pad = jnp.full((mpad - idx.shape[0],), fill, jnp.int32)
  full = jnp.concatenate([idx, pad])
  ph = mpad // parts
  out = [full[i * ph:(i + 1) * ph].reshape(_NW, ph // (_NW * _CH), _CH)
         for i in range(parts)]
  return out[0] if parts == 1 else out


def _pad_vals(v, mpad):
  pad = jnp.zeros((mpad - v.shape[0], v.shape[1]), v.dtype)
  return jnp.concatenate([v, pad], axis=0)


def _sc_gather(table, idx_padded, mpad):
  return _make_sc_gather(table.shape[1], mpad, table.shape[0])(
      table, idx_padded)


def _sc_segsum_lin(vals, dst_padded, mpad, nout):
  f = vals.shape[1]
  if vals.shape[0] != mpad:
    vals = _pad_vals(vals, mpad)
  zeros = jnp.zeros((nout // _NS, f), jnp.float32)
  return _make_sc_scatter_add(f, mpad, nout, False)(vals, dst_padded, zeros)


def _sc_segsum_gather(table_z, src_padded, dst_padded, mpad, nout):
  f = table_z.shape[1]
  zeros = jnp.zeros((nout // _NS, f), jnp.float32)
  return _make_sc_scatter_add(f, mpad, nout, True, table_z.shape[0])(
      table_z, src_padded, dst_padded, zeros)


def _edge_messages(ea, xs, w1, b1, w2, b2, r, s_mat, fo, half):
  e = xs.shape[0]
  be = 1280
  fi = xs.shape[1]
  kw = w2.shape[1]
  hoff = half * (e // be)

  def body(ea_ref, xs_ref, w1_ref, b1_ref, w2_ref, b2_ref, r_ref, s_ref,
           out_ref):
    h = jnp.maximum(ea_ref[...] @ w1_ref[...] + b1_ref[...], 0.0)
    t = jnp.maximum(h @ w2_ref[...] + b2_ref[...], 0.0)
    xr = xs_ref[...] @ r_ref[...]
    out_ref[...] = (xr * t) @ s_ref[...]

  return pl.pallas_call(
      body,
      grid=(e // be,),
      in_specs=[
          pl.BlockSpec((be, _B_IN), lambda i: (i + hoff, 0)),
          pl.BlockSpec((be, fi), lambda i: (i, 0)),
          pl.BlockSpec((_B_IN, 128), lambda i: (0, 0)),
          pl.BlockSpec((1, 128), lambda i: (0, 0)),
          pl.BlockSpec((128, kw), lambda i: (0, 0)),
          pl.BlockSpec((1, kw), lambda i: (0, 0)),
          pl.BlockSpec((fi, kw), lambda i: (0, 0)),
          pl.BlockSpec((kw, fo), lambda i: (0, 0)),
      ],
      out_specs=pl.BlockSpec((be, fo), lambda i: (i, 0)),
      out_shape=jax.ShapeDtypeStruct((e, fo), jnp.float32),
  )(ea, xs, w1, b1, w2, b2, r, s_mat)


def _node_update(x, w, b, aggp0, aggp1):
  n = x.shape[0]
  fo = w.shape[1]

  def body(x_ref, w_ref, b_ref, a0_ref, a1_ref, o_ref):
    agg = (a0_ref[0] + a0_ref[1]) + (a1_ref[0] + a1_ref[1])
    o_ref[...] = jnp.maximum(x_ref[...] @ w_ref[...] + agg + b_ref[...], 0.0)

  return pl.pallas_call(
      body,
      out_shape=jax.ShapeDtypeStruct((n, fo), jnp.float32),
  )(x, w, b, aggp0, aggp1)


def _graph_dense(x, aggp, w_rel, w_root, b):
  n, fi = x.shape
  fo = w_rel.shape[1]
  bn = 2000

  def body(x_ref, a_ref, wr_ref, wx_ref, b_ref, o_ref):
    agg = a_ref[0] + a_ref[1]
    o_ref[...] = jnp.maximum(
        agg @ wr_ref[...] + x_ref[...] @ wx_ref[...] + b_ref[...], 0.0)

  return pl.pallas_call(
      body,
      grid=(n // bn,),
      in_specs=[
          pl.BlockSpec((bn, fi), lambda i: (i, 0)),
          pl.BlockSpec((_NC, bn, fi), lambda i: (0, i, 0)),
          pl.BlockSpec((fi, fo), lambda i: (0, 0)),
          pl.BlockSpec((fi, fo), lambda i: (0, 0)),
          pl.BlockSpec((1, fo), lambda i: (0, 0)),
      ],
      out_specs=pl.BlockSpec((bn, fo), lambda i: (i, 0)),
      out_shape=jax.ShapeDtypeStruct((n, fo), jnp.float32),
  )(x, aggp, w_rel, w_root, b)


def _avgpool_finish(poolp, cntp):
  n = poolp.shape[1]
  bn = 2000

  def body(p_ref, c_ref, o_ref):
    ssum = p_ref[0] + p_ref[1]
    cnt = jnp.maximum(c_ref[0][:, 0:1] + c_ref[1][:, 0:1], 1.0)
    o_ref[...] = ssum / cnt

  return pl.pallas_call(
      body,
      grid=(n // bn,),
      in_specs=[
          pl.BlockSpec((_NC, bn, 64), lambda i: (0, i, 0)),
          pl.BlockSpec((_NC, bn, 16), lambda i: (0, i, 0)),
      ],
      out_specs=pl.BlockSpec((bn, 64), lambda i: (i, 0)),
      out_shape=jax.ShapeDtypeStruct((n, 64), jnp.float32),
  )(poolp, cntp)


def _graph_dense_c4(xa, iso, aggp, isoaggp, wr_a, wr_b, wx_a, wx_b, b):
  n = xa.shape[0]
  fo = wr_a.shape[1]
  bn = 2000

  def body(x_ref, i_ref, a_ref, ia_ref, wra_ref, wrb_ref, wxa_ref, wxb_ref,
           b_ref, o_ref):
    agg = a_ref[0] + a_ref[1]
    iagg = ia_ref[0] + ia_ref[1]
    o_ref[...] = jnp.maximum(
        agg @ wra_ref[...] + iagg @ wrb_ref[...]
        + x_ref[...] @ wxa_ref[...] + i_ref[...] @ wxb_ref[...]
        + b_ref[...], 0.0)

  return pl.pallas_call(
      body,
      grid=(n // bn,),
      in_specs=[
          pl.BlockSpec((bn, 64), lambda i: (i, 0)),
          pl.BlockSpec((bn, _NUM_I2), lambda i: (i, 0)),
          pl.BlockSpec((_NC, bn, 64), lambda i: (0, i, 0)),
          pl.BlockSpec((_NC, bn, _NUM_I2), lambda i: (0, i, 0)),
          pl.BlockSpec((64, fo), lambda i: (0, 0)),
          pl.BlockSpec((_NUM_I2, fo), lambda i: (0, 0)),
          pl.BlockSpec((64, fo), lambda i: (0, 0)),
          pl.BlockSpec((_NUM_I2, fo), lambda i: (0, 0)),
          pl.BlockSpec((1, fo), lambda i: (0, 0)),
      ],
      out_specs=pl.BlockSpec((bn, fo), lambda i: (i, 0)),
      out_shape=jax.ShapeDtypeStruct((n, fo), jnp.float32),
  )(xa, iso, aggp, isoaggp, wr_a, wr_b, wx_a, wx_b, b)


def _head(x1gp, x2gp, u0, u1, w0, b0, w1, b1, wl, bl, pl0, pl1):
  eps = 1e-7
  temp = 0.1

  def body(x1_ref, x2_ref, u0_ref, u1_ref, w0_ref, b0_ref, w1_ref, b1_ref,
           wl_ref, bl_ref, p0_ref, p1_ref, o_ref, reg_ref):
    z = jnp.concatenate(
        [x1_ref[0] + x1_ref[1], x2_ref[0] + x2_ref[1]], axis=1)

    def drop(zin, u, p):
      dp = (jnp.log(p + eps) - jnp.log(1.0 - p + eps)
            + jnp.log(u + eps) - jnp.log(1.0 - u + eps))
      dp = 1.0 / (1.0 + jnp.exp(-dp / temp))
      return zin * (1.0 - dp) / (1.0 - p)

    def regterm(w, b, p, d):
      sum_sq = jnp.sum(w * w) + jnp.sum(b * b)
      ent = p * jnp.log(p + eps) + (1.0 - p) * jnp.log(1.0 - p + eps)
      return _W_REG * sum_sq / (1.0 - p) + _D_REG * d * ent

    p0 = 1.0 / (1.0 + jnp.exp(-p0_ref[0, 0]))
    p1 = 1.0 / (1.0 + jnp.exp(-p1_ref[0, 0]))
    z = jnp.maximum(drop(z, u0_ref[...], p0) @ w0_ref[...] + b0_ref[...], 0.0)
    z = jnp.maximum(drop(z, u1_ref[...], p1) @ w1_ref[...] + b1_ref[...], 0.0)
    o_ref[...] = z @ wl_ref[...] + bl_ref[...]
    reg = (regterm(w0_ref[...], b0_ref[...], p0, 2.0 * _DIM)
           + regterm(w1_ref[...], b1_ref[...], p1, 1.0 * _DIM))
    reg_ref[...] = jnp.broadcast_to(reg, (1, 1))

  return pl.pallas_call(
      body,
      out_shape=(
          jax.ShapeDtypeStruct((_G, 2), jnp.float32),
          jax.ShapeDtypeStruct((1, 1), jnp.float32),
      ),
  )(x1gp, x2gp, u0, u1, w0, b0, w1, b1, wl, bl, pl0, pl1)


def _expand_reduce_mats(fi, fo):
  k = fi * fo
  ii = jnp.arange(k, dtype=jnp.int32) // fo
  oo = jnp.arange(k, dtype=jnp.int32) % fo
  r = (jnp.arange(fi, dtype=jnp.int32)[:, None] == ii[None, :])
  s = (oo[:, None] == jnp.arange(fo, dtype=jnp.int32)[None, :])
  return r.astype(jnp.float32), s.astype(jnp.float32)


def kernel(x, edge_index, edge_attr, batch, assignment_index_2, iso_type_2,
           edge_index_2, batch_2, params):
  p = params
  f32 = jnp.float32
  row = lambda a: a.reshape(1, -1).astype(f32)

  src = edge_index[0]
  dst = edge_index[1]
  eh = _EPAD // 2
  src_h = _pad_idx(src, _EPAD, 0, 2)
  dst_h = _pad_idx(dst, _EPAD, _N, 2)
  ea_pad = _pad_vals(edge_attr, _EPAD)

  r0, s0 = _expand_reduce_mats(_NUM_FEAT, _DIM // 2)
  r1, s1 = _expand_reduce_mats(_DIM // 2, _DIM)

  a_src = _pad_idx(assignment_index_2[0], _APAD, 0)
  a_dst = _pad_idx(assignment_index_2[1], _APAD, _N2)
  s2_g = _pad_idx(edge_index_2[0], _EPAD, 0)
  d2_s = _pad_idx(edge_index_2[1], _EPAD, _N2)

  def nnconv(xin, w1, b1, w2, b2, r, s_mat, fo, root, bconv):
    aggp = []
    for h in range(2):
      xs = _sc_gather(xin, src_h[h], eh)
      msg = _edge_messages(ea_pad, xs, w1, b1, w2, b2, r, s_mat, fo, h)
      aggp.append(_sc_segsum_lin(msg, dst_h[h], eh, _N))
    return _node_update(xin, root, bconv, aggp[0], aggp[1])

  x1 = nnconv(x, p['nn0_W1'], row(p['nn0_b1']), p['nn0_W2'], row(p['nn0_b2']),
              r0, s0, _DIM // 2, p['conv0_root'], row(p['conv0_b']))
  cntp = _sc_segsum_lin(jnp.ones((_APAD, 16), f32), a_dst, _APAD, _N2)
  isoaggp = _sc_segsum_gather(iso_type_2, s2_g, d2_s, _EPAD, _N2)
  x2 = nnconv(x1, p['nn1_W1'], row(p['nn1_b1']), p['nn1_W2'], row(p['nn1_b2']),
              r1, s1, _DIM, p['conv1_root'], row(p['conv1_b']))

  batch_s = _pad_idx(batch, _NPAD, _G)
  x1gp = _sc_segsum_lin(x2, batch_s, _NPAD, _G)

  poolp = _sc_segsum_gather(x2, a_src, a_dst, _APAD, _N2)
  x2avg = _avgpool_finish(poolp, cntp)

  c4aggp = _sc_segsum_gather(x2avg, s2_g, d2_s, _EPAD, _N2)
  x2b = _graph_dense_c4(x2avg, iso_type_2, c4aggp, isoaggp,
                        p['c4_Wrel'][:64], p['c4_Wrel'][64:],
                        p['c4_Wroot'][:64], p['c4_Wroot'][64:],
                        row(p['c4_b']))
  c5aggp = _sc_segsum_gather(x2b, s2_g, d2_s, _EPAD, _N2)
  x2c = _graph_dense(x2b, c5aggp, p['c5_Wrel'], p['c5_Wroot'],
                     row(p['c5_b']))

  batch2_s = _pad_idx(batch_2, _NPAD, _G)
  x2gp = _sc_segsum_lin(x2c, batch2_s, _NPAD, _G)

  u0 = jax.random.uniform(jax.random.key(101), (_G, 2 * _DIM), dtype=f32)
  u1 = jax.random.uniform(jax.random.key(202), (_G, _DIM), dtype=f32)
  wl = jnp.concatenate([p['mu_W'], p['lv_W']], axis=1)
  bl = jnp.concatenate([p['mu_b'], p['lv_b']]).reshape(1, 2)
  out, reg = _head(x1gp, x2gp, u0, u1,
                   p['fc0_W'], row(p['fc0_b']), p['fc1_W'], row(p['fc1_b']),
                   wl, bl, p['p_logit0'].reshape(1, 1),
                   p['p_logit1'].reshape(1, 1))
  mean = out[:, 0:1]
  log_var = out[:, 1:2]
  return mean, log_var, reg.reshape(())


# speedup vs baseline: 1.0913x; 1.0913x over previous
import functools
import jax
import jax.numpy as jnp
from jax import lax
from jax.experimental import pallas as pl
from jax.experimental.pallas import tpu as pltpu
from jax.experimental.pallas import tpu_sc as plsc

_N = 10000
_E = 80000
_A = 20000
_N2 = 10000
_E2 = 80000
_G = 512
_NUM_FEAT = 64
_B_IN = 16
_DIM = 64
_NUM_I2 = 32
_W_REG = 1e-6
_D_REG = 1e-5

_NC = 2
_NS = 16
_NW = _NC * _NS
_CH = 128

_EPAD = 81920
_NPAD = 12288
_APAD = 20480


def _group_size(ncw):
  for k in (5, 4, 3, 2):
    if ncw % k == 0:
      return k
  return 1


@functools.lru_cache(maxsize=None)
def _make_sc_gather(f, mpad, vrows):
  ncw = mpad // (_NW * _CH)
  kk = 1
  for cand in (10, 5, 4, 3, 2):
    if ncw % cand:
      continue
    if _NS * (ncw * _CH + cand * _CH * f) + vrows * f < 2_050_000:
      kk = cand
      break
  mesh = plsc.VectorSubcoreMesh(core_axis_name="c", subcore_axis_name="s")

  def body(table_hbm, idx_hbm, out_hbm, idx_v, buf_v, table_spm, sem):
    c = lax.axis_index("c")
    s = lax.axis_index("s")
    wid = s * _NC + c
    trows = vrows // _NS
    pltpu.sync_copy(table_hbm.at[pl.ds(s * trows, trows)],
                    table_spm.at[pl.ds(s * trows, trows)])
    pltpu.sync_copy(idx_hbm.at[wid], idx_v)
    plsc.subcore_barrier()
    base = wid * (ncw * _CH)

    def group(g, carry):
      j0 = g * kk
      handles = [
          pltpu.async_copy(table_spm.at[idx_v.at[j0 + b]],
                           buf_v.at[pl.ds(b * _CH, _CH)], sem)
          for b in range(kk)
      ]
      for b, h in enumerate(handles):
        h.wait()
        pltpu.sync_copy(buf_v.at[pl.ds(b * _CH, _CH)],
                        out_hbm.at[pl.ds(base + (j0 + b) * _CH, _CH)])
      return carry

    lax.fori_loop(0, ncw // kk, group, 0)

  return pl.kernel(
      body,
      out_type=jax.ShapeDtypeStruct((mpad, f), jnp.float32),
      mesh=mesh,
      compiler_params=pltpu.CompilerParams(use_tc_tiling_on_sc=False),
      scratch_types=[
          pltpu.VMEM((ncw, _CH), jnp.int32),
          pltpu.VMEM((kk * _CH, f), jnp.float32),
          pltpu.VMEM_SHARED((vrows, f), jnp.float32),
          pltpu.SemaphoreType.DMA,
      ],
  )


@functools.lru_cache(maxsize=None)
def _make_sc_scatter_add(f, mpad, nout, indirect, vrows=0):
  ncw = mpad // (_NW * _CH)
  rows = nout // _NS
  tab_words = (vrows * f) if indirect else 0
  kk = 1
  for cand in (10, 5, 4, 3, 2):
    if ncw % cand:
      continue
    spmem_words = (_NS * (2 * ncw * _CH + cand * _CH * f)
                   + (nout + 8) * f + tab_words)
    if spmem_words < 2_050_000:
      kk = cand
      break
  mesh = plsc.VectorSubcoreMesh(core_axis_name="c", subcore_axis_name="s")

  def run(vals_hbm, src_hbm, dst_hbm, zeros_hbm, out_hbm,
          src_v, dst_v, buf_v, accum, table_spm, sem):
    c = lax.axis_index("c")
    s = lax.axis_index("s")
    wid = s * _NC + c
    pltpu.sync_copy(zeros_hbm, accum.at[pl.ds(s * rows, rows)])
    pltpu.sync_copy(zeros_hbm.at[pl.ds(0, 8)], accum.at[pl.ds(nout, 8)])
    if indirect:
      trows = vrows // _NS
      pltpu.sync_copy(vals_hbm.at[pl.ds(s * trows, trows)],
                      table_spm.at[pl.ds(s * trows, trows)])
      pltpu.sync_copy(src_hbm.at[wid], src_v)
    plsc.subcore_barrier()
    pltpu.sync_copy(dst_hbm.at[wid], dst_v)
    base = wid * (ncw * _CH)

    def group(g, carry):
      j0 = g * kk
      if indirect:
        handles = [
            pltpu.async_copy(table_spm.at[src_v.at[j0 + b]],
                             buf_v.at[pl.ds(b * _CH, _CH)], sem)
            for b in range(kk)
        ]
      else:
        handles = [
            pltpu.async_copy(vals_hbm.at[pl.ds(base + (j0 + b) * _CH, _CH)],
                             buf_v.at[pl.ds(b * _CH, _CH)], sem)
            for b in range(kk)
        ]
      for b, h in enumerate(handles):
        h.wait()
        pltpu.sync_copy(buf_v.at[pl.ds(b * _CH, _CH)],
                        accum.at[dst_v.at[j0 + b]], add=True)
      return carry

    lax.fori_loop(0, ncw // kk, group, 0)
    plsc.subcore_barrier()
    pltpu.sync_copy(accum.at[pl.ds(s * rows, rows)],
                    out_hbm.at[c, pl.ds(s * rows, rows)])

  scratch = [
      pltpu.VMEM((ncw, _CH), jnp.int32),
      pltpu.VMEM((ncw, _CH), jnp.int32),
      pltpu.VMEM((kk * _CH, f), jnp.float32),
      pltpu.VMEM_SHARED((nout + 8, f), jnp.float32),
  ]
  if indirect:
    body = run
    scratch.append(pltpu.VMEM_SHARED((vrows, f), jnp.float32))
  else:
    def body(vals_hbm, dst_hbm, zeros_hbm, out_hbm,
             src_v, dst_v, buf_v, accum, sem):
      run(vals_hbm, None, dst_hbm, zeros_hbm, out_hbm,
          src_v, dst_v, buf_v, accum, None, sem)
  scratch.append(pltpu.SemaphoreType.DMA)

  return pl.kernel(
      body,
      out_type=jax.ShapeDtypeStruct((_NC, nout, f), jnp.float32),
      mesh=mesh,
      compiler_params=pltpu.CompilerParams(use_tc_tiling_on_sc=False),
      scratch_types=scratch,
  )


def _pad_idx(idx, mpad, fill, parts=1):
  pad = jnp.full((mpad - idx.shape[0],), fill, jnp.int32)
  full = jnp.concatenate([idx, pad])
  ph = mpad // parts
  out = [full[i * ph:(i + 1) * ph].reshape(_NW, ph // (_NW * _CH), _CH)
         for i in range(parts)]
  return out[0] if parts == 1 else out


def _pad_vals(v, mpad):
  pad = jnp.zeros((mpad - v.shape[0], v.shape[1]), v.dtype)
  return jnp.concatenate([v, pad], axis=0)


def _sc_gather(table, idx_padded, mpad):
  return _make_sc_gather(table.shape[1], mpad, table.shape[0])(
      table, idx_padded)


def _sc_segsum_lin(vals, dst_padded, mpad, nout):
  f = vals.shape[1]
  if vals.shape[0] != mpad:
    vals = _pad_vals(vals, mpad)
  zeros = jnp.zeros((nout // _NS, f), jnp.float32)
  return _make_sc_scatter_add(f, mpad, nout, False)(vals, dst_padded, zeros)


def _sc_segsum_gather(table_z, src_padded, dst_padded, mpad, nout):
  f = table_z.shape[1]
  zeros = jnp.zeros((nout // _NS, f), jnp.float32)
  return _make_sc_scatter_add(f, mpad, nout, True, table_z.shape[0])(
      table_z, src_padded, dst_padded, zeros)


def _edge_messages(ea, xs, w1, b1, w2, b2, r, s_mat, fo, half):
  e = xs.shape[0]
  be = 1280
  fi = xs.shape[1]
  kw = w2.shape[1]
  hoff = half * (e // be)

  def body(ea_ref, xs_ref, w1_ref, b1_ref, w2_ref, b2_ref, r_ref, s_ref,
           out_ref):
    h = jnp.maximum(ea_ref[...] @ w1_ref[...] + b1_ref[...], 0.0)
    t = jnp.maximum(h @ w2_ref[...] + b2_ref[...], 0.0)
    xr = xs_ref[...] @ r_ref[...]
    out_ref[...] = (xr * t) @ s_ref[...]

  return pl.pallas_call(
      body,
      grid=(e // be,),
      in_specs=[
          pl.BlockSpec((be, _B_IN), lambda i: (i + hoff, 0)),
          pl.BlockSpec((be, fi), lambda i: (i, 0)),
          pl.BlockSpec((_B_IN, 128), lambda i: (0, 0)),
          pl.BlockSpec((1, 128), lambda i: (0, 0)),
          pl.BlockSpec((128, kw), lambda i: (0, 0)),
          pl.BlockSpec((1, kw), lambda i: (0, 0)),
          pl.BlockSpec((fi, kw), lambda i: (0, 0)),
          pl.BlockSpec((kw, fo), lambda i: (0, 0)),
      ],
      out_specs=pl.BlockSpec((be, fo), lambda i: (i, 0)),
      out_shape=jax.ShapeDtypeStruct((e, fo), jnp.float32),
  )(ea, xs, w1, b1, w2, b2, r, s_mat)


def _node_update(x, w, b, aggp0, aggp1, fo):
  n = x.shape[0]

  def body(x_ref, w_ref, b_ref, a0_ref, a1_ref, o_ref):
    agg = ((a0_ref[0] + a0_ref[1]) + (a1_ref[0] + a1_ref[1]))[:, :fo]
    o_ref[...] = jnp.maximum(x_ref[...] @ w_ref[...] + agg + b_ref[...], 0.0)

  return pl.pallas_call(
      body,
      out_shape=jax.ShapeDtypeStruct((n, fo), jnp.float32),
  )(x, w, b, aggp0, aggp1)


def _graph_dense(x, aggp, w_rel, w_root, b):
  n, fi = x.shape
  fo = w_rel.shape[1]
  bn = 2000

  def body(x_ref, a_ref, wr_ref, wx_ref, b_ref, o_ref):
    agg = a_ref[0] + a_ref[1]
    o_ref[...] = jnp.maximum(
        agg @ wr_ref[...] + x_ref[...] @ wx_ref[...] + b_ref[...], 0.0)

  return pl.pallas_call(
      body,
      grid=(n // bn,),
      in_specs=[
          pl.BlockSpec((bn, fi), lambda i: (i, 0)),
          pl.BlockSpec((_NC, bn, fi), lambda i: (0, i, 0)),
          pl.BlockSpec((fi, fo), lambda i: (0, 0)),
          pl.BlockSpec((fi, fo), lambda i: (0, 0)),
          pl.BlockSpec((1, fo), lambda i: (0, 0)),
      ],
      out_specs=pl.BlockSpec((bn, fo), lambda i: (i, 0)),
      out_shape=jax.ShapeDtypeStruct((n, fo), jnp.float32),
  )(x, aggp, w_rel, w_root, b)


def _avgpool_finish(poolp, cntp):
  n = poolp.shape[1]
  bn = 2000

  def body(p_ref, c_ref, o_ref):
    ssum = p_ref[0] + p_ref[1]
    cnt = jnp.maximum(c_ref[0][:, 0:1] + c_ref[1][:, 0:1], 1.0)
    o_ref[...] = ssum / cnt

  return pl.pallas_call(
      body,
      grid=(n // bn,),
      in_specs=[
          pl.BlockSpec((_NC, bn, 64), lambda i: (0, i, 0)),
          pl.BlockSpec((_NC, bn, 16), lambda i: (0, i, 0)),
      ],
      out_specs=pl.BlockSpec((bn, 64), lambda i: (i, 0)),
      out_shape=jax.ShapeDtypeStruct((n, 64), jnp.float32),
  )(poolp, cntp)


def _graph_dense_c4(xa, iso, aggp, isoaggp, wr_a, wr_b, wx_a, wx_b, b):
  n = xa.shape[0]
  fo = wr_a.shape[1]
  bn = 2000

  def body(x_ref, i_ref, a_ref, ia_ref, wra_ref, wrb_ref, wxa_ref, wxb_ref,
           b_ref, o_ref):
    agg = a_ref[0] + a_ref[1]
    iagg = ia_ref[0] + ia_ref[1]
    o_ref[...] = jnp.maximum(
        agg @ wra_ref[...] + iagg @ wrb_ref[...]
        + x_ref[...] @ wxa_ref[...] + i_ref[...] @ wxb_ref[...]
        + b_ref[...], 0.0)

  return pl.pallas_call(
      body,
      grid=(n // bn,),
      in_specs=[
          pl.BlockSpec((bn, 64), lambda i: (i, 0)),
          pl.BlockSpec((bn, _NUM_I2), lambda i: (i, 0)),
          pl.BlockSpec((_NC, bn, 64), lambda i: (0, i, 0)),
          pl.BlockSpec((_NC, bn, _NUM_I2), lambda i: (0, i, 0)),
          pl.BlockSpec((64, fo), lambda i: (0, 0)),
          pl.BlockSpec((_NUM_I2, fo), lambda i: (0, 0)),
          pl.BlockSpec((64, fo), lambda i: (0, 0)),
          pl.BlockSpec((_NUM_I2, fo), lambda i: (0, 0)),
          pl.BlockSpec((1, fo), lambda i: (0, 0)),
      ],
      out_specs=pl.BlockSpec((bn, fo), lambda i: (i, 0)),
      out_shape=jax.ShapeDtypeStruct((n, fo), jnp.float32),
  )(xa, iso, aggp, isoaggp, wr_a, wr_b, wx_a, wx_b, b)


def _head(x1gp, x2gp, u0, u1, w0, b0, w1, b1, wl, bl, pl0, pl1):
  eps = 1e-7
  temp = 0.1

  def body(x1_ref, x2_ref, u0_ref, u1_ref, w0_ref, b0_ref, w1_ref, b1_ref,
           wl_ref, bl_ref, p0_ref, p1_ref, o_ref, reg_ref):
    z = jnp.concatenate(
        [x1_ref[0] + x1_ref[1], x2_ref[0] + x2_ref[1]], axis=1)

    def drop(zin, u, p):
      dp = (jnp.log(p + eps) - jnp.log(1.0 - p + eps)
            + jnp.log(u + eps) - jnp.log(1.0 - u + eps))
      dp = 1.0 / (1.0 + jnp.exp(-dp / temp))
      return zin * (1.0 - dp) / (1.0 - p)

    def regterm(w, b, p, d):
      sum_sq = jnp.sum(w * w) + jnp.sum(b * b)
      ent = p * jnp.log(p + eps) + (1.0 - p) * jnp.log(1.0 - p + eps)
      return _W_REG * sum_sq / (1.0 - p) + _D_REG * d * ent

    p0 = 1.0 / (1.0 + jnp.exp(-p0_ref[0, 0]))
    p1 = 1.0 / (1.0 + jnp.exp(-p1_ref[0, 0]))
    z = jnp.maximum(drop(z, u0_ref[...], p0) @ w0_ref[...] + b0_ref[...], 0.0)
    z = jnp.maximum(drop(z, u1_ref[...], p1) @ w1_ref[...] + b1_ref[...], 0.0)
    o_ref[...] = z @ wl_ref[...] + bl_ref[...]
    reg = (regterm(w0_ref[...], b0_ref[...], p0, 2.0 * _DIM)
           + regterm(w1_ref[...], b1_ref[...], p1, 1.0 * _DIM))
    reg_ref[...] = jnp.broadcast_to(reg, (1, 1))

  return pl.pallas_call(
      body,
      out_shape=(
          jax.ShapeDtypeStruct((_G, 2), jnp.float32),
          jax.ShapeDtypeStruct((1, 1), jnp.float32),
      ),
  )(x1gp, x2gp, u0, u1, w0, b0, w1, b1, wl, bl, pl0, pl1)


def _expand_reduce_mats(fi, fo, fo_pad):
  k = fi * fo
  ii = jnp.arange(k, dtype=jnp.int32) // fo
  oo = jnp.arange(k, dtype=jnp.int32) % fo
  r = (jnp.arange(128, dtype=jnp.int32)[:, None] == ii[None, :])
  s = (oo[:, None] == jnp.arange(fo_pad, dtype=jnp.int32)[None, :])
  return r.astype(jnp.float32), s.astype(jnp.float32)


def kernel(x, edge_index, edge_attr, batch, assignment_index_2, iso_type_2,
           edge_index_2, batch_2, params):
  p = params
  f32 = jnp.float32
  row = lambda a: a.reshape(1, -1).astype(f32)

  src = edge_index[0]
  dst = edge_index[1]
  eh = _EPAD // 2
  src_h = _pad_idx(src, _EPAD, 0, 2)
  dst_h = _pad_idx(dst, _EPAD, _N, 2)
  ea_pad = _pad_vals(edge_attr, _EPAD)

  r0, s0 = _expand_reduce_mats(_NUM_FEAT, _DIM // 2, 128)
  r1, s1 = _expand_reduce_mats(_DIM // 2, _DIM, 128)

  a_src = _pad_idx(assignment_index_2[0], _APAD, 0)
  a_dst = _pad_idx(assignment_index_2[1], _APAD, _N2)
  s2_g = _pad_idx(edge_index_2[0], _EPAD, 0)
  d2_s = _pad_idx(edge_index_2[1], _EPAD, _N2)

  def nnconv(xin, w1, b1, w2, b2, r, s_mat, fo, root, bconv):
    aggp = []
    xpad = jnp.concatenate(
        [xin, jnp.zeros((xin.shape[0], 128 - xin.shape[1]), f32)], axis=1)
    for h in range(2):
      xs = _sc_gather(xpad, src_h[h], eh)
      msg = _edge_messages(ea_pad, xs, w1, b1, w2, b2, r, s_mat, 128, h)
      aggp.append(_sc_segsum_lin(msg, dst_h[h], eh, _N))
    return _node_update(xin, root, bconv, aggp[0], aggp[1], fo)

  x1 = nnconv(x, p['nn0_W1'], row(p['nn0_b1']), p['nn0_W2'], row(p['nn0_b2']),
              r0, s0, _DIM // 2, p['conv0_root'], row(p['conv0_b']))
  cntp = _sc_segsum_lin(jnp.ones((_APAD, 16), f32), a_dst, _APAD, _N2)
  isoaggp = _sc_segsum_gather(iso_type_2, s2_g, d2_s, _EPAD, _N2)
  x2 = nnconv(x1, p['nn1_W1'], row(p['nn1_b1']), p['nn1_W2'], row(p['nn1_b2']),
              r1, s1, _DIM, p['conv1_root'], row(p['conv1_b']))

  batch_s = _pad_idx(batch, _NPAD, _G)
  x1gp = _sc_segsum_lin(x2, batch_s, _NPAD, _G)

  poolp = _sc_segsum_gather(x2, a_src, a_dst, _APAD, _N2)
  x2avg = _avgpool_finish(poolp, cntp)

  c4aggp = _sc_segsum_gather(x2avg, s2_g, d2_s, _EPAD, _N2)
  x2b = _graph_dense_c4(x2avg, iso_type_2, c4aggp, isoaggp,
                        p['c4_Wrel'][:64], p['c4_Wrel'][64:],
                        p['c4_Wroot'][:64], p['c4_Wroot'][64:],
                        row(p['c4_b']))
  c5aggp = _sc_segsum_gather(x2b, s2_g, d2_s, _EPAD, _N2)
  x2c = _graph_dense(x2b, c5aggp, p['c5_Wrel'], p['c5_Wroot'],
                     row(p['c5_b']))

  batch2_s = _pad_idx(batch_2, _NPAD, _G)
  x2gp = _sc_segsum_lin(x2c, batch2_s, _NPAD, _G)

  u0 = jax.random.uniform(jax.random.key(101), (_G, 2 * _DIM), dtype=f32)
  u1 = jax.random.uniform(jax.random.key(202), (_G, _DIM), dtype=f32)
  wl = jnp.concatenate([p['mu_W'], p['lv_W']], axis=1)
  bl = jnp.concatenate([p['mu_b'], p['lv_b']]).reshape(1, 2)
  out, reg = _head(x1gp, x2gp, u0, u1,
                   p['fc0_W'], row(p['fc0_b']), p['fc1_W'], row(p['fc1_b']),
                   wl, bl, p['p_logit0'].reshape(1, 1),
                   p['p_logit1'].reshape(1, 1))
  mean = out[:, 0:1]
  log_var = out[:, 1:2]
  return mean, log_var, reg.reshape(())


# speedup vs baseline: 1.0928x; 1.0014x over previous
import functools
import jax
import jax.numpy as jnp
from jax import lax
from jax.experimental import pallas as pl
from jax.experimental.pallas import tpu as pltpu
from jax.experimental.pallas import tpu_sc as plsc

_N = 10000
_E = 80000
_A = 20000
_N2 = 10000
_E2 = 80000
_G = 512
_NUM_FEAT = 64
_B_IN = 16
_DIM = 64
_NUM_I2 = 32
_W_REG = 1e-6
_D_REG = 1e-5

_NC = 2
_NS = 16
_NW = _NC * _NS
_CH = 128

_EPAD = 81920
_NPAD = 12288
_APAD = 20480


@functools.lru_cache(maxsize=None)
def _make_sc_gather(f, mpad, vrows):
  ncw = mpad // (_NW * _CH)
  kk = 1
  for cand in (10, 5, 4, 3, 2):
    if ncw % cand:
      continue
    if _NS * (ncw * _CH + cand * _CH * f) + vrows * f < 2_050_000:
      kk = cand
      break
  mesh = plsc.VectorSubcoreMesh(core_axis_name="c", subcore_axis_name="s")

  def body(table_hbm, idx_hbm, out_hbm, idx_v, buf_v, table_spm, sem):
    c = lax.axis_index("c")
    s = lax.axis_index("s")
    wid = s * _NC + c
    trows = vrows // _NS
    pltpu.sync_copy(table_hbm.at[pl.ds(s * trows, trows)],
                    table_spm.at[pl.ds(s * trows, trows)])
    pltpu.sync_copy(idx_hbm.at[wid], idx_v)
    plsc.subcore_barrier()
    base = wid * (ncw * _CH)

    def group(g, carry):
      j0 = g * kk
      handles = [
          pltpu.async_copy(table_spm.at[idx_v.at[j0 + b]],
                           buf_v.at[pl.ds(b * _CH, _CH)], sem)
          for b in range(kk)
      ]
      for b, h in enumerate(handles):
        h.wait()
        pltpu.sync_copy(buf_v.at[pl.ds(b * _CH, _CH)],
                        out_hbm.at[pl.ds(base + (j0 + b) * _CH, _CH)])
      return carry

    lax.fori_loop(0, ncw // kk, group, 0)

  return pl.kernel(
      body,
      out_type=jax.ShapeDtypeStruct((mpad, f), jnp.float32),
      mesh=mesh,
      compiler_params=pltpu.CompilerParams(use_tc_tiling_on_sc=False),
      scratch_types=[
          pltpu.VMEM((ncw, _CH), jnp.int32),
          pltpu.VMEM((kk * _CH, f), jnp.float32),
          pltpu.VMEM_SHARED((vrows, f), jnp.float32),
          pltpu.SemaphoreType.DMA,
      ],
  )


@functools.lru_cache(maxsize=None)
def _make_sc_scatter_add(f, mpad, nout, indirect, vrows=0):
  ncw = mpad // (_NW * _CH)
  rows = nout // _NS
  tab_words = (vrows * f) if indirect else 0
  kk = 1
  for cand in (10, 5, 4, 3, 2):
    if ncw % cand:
      continue
    spmem_words = (_NS * (2 * ncw * _CH + cand * _CH * f)
                   + (nout + 8) * f + tab_words)
    if spmem_words < 2_050_000:
      kk = cand
      break
  mesh = plsc.VectorSubcoreMesh(core_axis_name="c", subcore_axis_name="s")

  def run(vals_hbm, src_hbm, dst_hbm, zeros_hbm, out_hbm,
          src_v, dst_v, buf_v, accum, table_spm, sem):
    c = lax.axis_index("c")
    s = lax.axis_index("s")
    wid = s * _NC + c
    pltpu.sync_copy(zeros_hbm, accum.at[pl.ds(s * rows, rows)])
    pltpu.sync_copy(zeros_hbm.at[pl.ds(0, 8)], accum.at[pl.ds(nout, 8)])
    if indirect:
      trows = vrows // _NS
      pltpu.sync_copy(vals_hbm.at[pl.ds(s * trows, trows)],
                      table_spm.at[pl.ds(s * trows, trows)])
      pltpu.sync_copy(src_hbm.at[wid], src_v)
    plsc.subcore_barrier()
    pltpu.sync_copy(dst_hbm.at[wid], dst_v)
    base = wid * (ncw * _CH)

    def group(g, carry):
      j0 = g * kk
      if indirect:
        handles = [
            pltpu.async_copy(table_spm.at[src_v.at[j0 + b]],
                             buf_v.at[pl.ds(b * _CH, _CH)], sem)
            for b in range(kk)
        ]
      else:
        handles = [
            pltpu.async_copy(vals_hbm.at[pl.ds(base + (j0 + b) * _CH, _CH)],
                             buf_v.at[pl.ds(b * _CH, _CH)], sem)
            for b in range(kk)
        ]
      for b, h in enumerate(handles):
        h.wait()
        pltpu.sync_copy(buf_v.at[pl.ds(b * _CH, _CH)],
                        accum.at[dst_v.at[j0 + b]], add=True)
      return carry

    lax.fori_loop(0, ncw // kk, group, 0)
    plsc.subcore_barrier()
    pltpu.sync_copy(accum.at[pl.ds(s * rows, rows)],
                    out_hbm.at[c, pl.ds(s * rows, rows)])

  scratch = [
      pltpu.VMEM((ncw, _CH), jnp.int32),
      pltpu.VMEM((ncw, _CH), jnp.int32),
      pltpu.VMEM((kk * _CH, f), jnp.float32),
      pltpu.VMEM_SHARED((nout + 8, f), jnp.float32),
  ]
  if indirect:
    body = run
    scratch.append(pltpu.VMEM_SHARED((vrows, f), jnp.float32))
  else:
    def body(vals_hbm, dst_hbm, zeros_hbm, out_hbm,
             src_v, dst_v, buf_v, accum, sem):
      run(vals_hbm, None, dst_hbm, zeros_hbm, out_hbm,
          src_v, dst_v, buf_v, accum, None, sem)
  scratch.append(pltpu.SemaphoreType.DMA)

  return pl.kernel(
      body,
      out_type=jax.ShapeDtypeStruct((_NC, nout, f), jnp.float32),
      mesh=mesh,
      compiler_params=pltpu.CompilerParams(use_tc_tiling_on_sc=False),
      scratch_types=scratch,
  )


def _pad_idx(idx, mpad, fill, parts=1):
  pad = jnp.full((mpad - idx.shape[0],), fill, jnp.int32)
  full = jnp.concatenate([idx, pad])
  ph = mpad // parts
  out = [full[i * ph:(i + 1) * ph].reshape(_NW, ph // (_NW * _CH), _CH)
         for i in range(parts)]
  return out[0] if parts == 1 else out


def _pad_vals(v, mpad):
  pad = jnp.zeros((mpad - v.shape[0], v.shape[1]), v.dtype)
  return jnp.concatenate([v, pad], axis=0)


def _sc_gather(table, idx_padded, mpad):
  return _make_sc_gather(table.shape[1], mpad, table.shape[0])(
      table, idx_padded)


def _sc_segsum_lin(vals, dst_padded, mpad, nout):
  f = vals.shape[1]
  if vals.shape[0] != mpad:
    vals = _pad_vals(vals, mpad)
  zeros = jnp.zeros((nout // _NS, f), jnp.float32)
  return _make_sc_scatter_add(f, mpad, nout, False)(vals, dst_padded, zeros)


def _sc_segsum_gather(table_z, src_padded, dst_padded, mpad, nout):
  f = table_z.shape[1]
  zeros = jnp.zeros((nout // _NS, f), jnp.float32)
  return _make_sc_scatter_add(f, mpad, nout, True, table_z.shape[0])(
      table_z, src_padded, dst_padded, zeros)


def _edge_messages(ea, xs, w1, b1, w2, b2, r, s_mat, fo, half):
  e = xs.shape[0]
  be = 1280
  fi = xs.shape[1]
  kw = w2.shape[1]
  hoff = half * (e // be)

  def body(ea_ref, xs_ref, w1_ref, b1_ref, w2_ref, b2_ref, r_ref, s_ref,
           out_ref):
    h = jnp.maximum(ea_ref[...] @ w1_ref[...] + b1_ref[...], 0.0)
    t = jnp.maximum(h @ w2_ref[...] + b2_ref[...], 0.0)
    xr = xs_ref[...] @ r_ref[...]
    out_ref[...] = (xr * t) @ s_ref[...]

  return pl.pallas_call(
      body,
      grid=(e // be,),
      in_specs=[
          pl.BlockSpec((be, _B_IN), lambda i: (i + hoff, 0)),
          pl.BlockSpec((be, fi), lambda i: (i, 0)),
          pl.BlockSpec((_B_IN, 128), lambda i: (0, 0)),
          pl.BlockSpec((1, 128), lambda i: (0, 0)),
          pl.BlockSpec((128, kw), lambda i: (0, 0)),
          pl.BlockSpec((1, kw), lambda i: (0, 0)),
          pl.BlockSpec((fi, kw), lambda i: (0, 0)),
          pl.BlockSpec((kw, fo), lambda i: (0, 0)),
      ],
      out_specs=pl.BlockSpec((be, fo), lambda i: (i, 0)),
      out_shape=jax.ShapeDtypeStruct((e, fo), jnp.float32),
  )(ea, xs, w1, b1, w2, b2, r, s_mat)


def _node_update(x, w, b, aggp0, aggp1, fo):
  n = x.shape[0]

  def body(x_ref, w_ref, b_ref, a0_ref, a1_ref, o_ref):
    agg = ((a0_ref[0] + a0_ref[1]) + (a1_ref[0] + a1_ref[1]))[:, :fo]
    o_ref[...] = jnp.maximum(x_ref[...] @ w_ref[...] + agg + b_ref[...], 0.0)

  return pl.pallas_call(
      body,
      out_shape=jax.ShapeDtypeStruct((n, fo), jnp.float32),
  )(x, w, b, aggp0, aggp1)


def _graph_dense(x, aggp, w_rel, w_root, b):
  n, fi = x.shape
  fo = w_rel.shape[1]
  bn = 2000

  def body(x_ref, a_ref, wr_ref, wx_ref, b_ref, o_ref):
    agg = a_ref[0] + a_ref[1]
    o_ref[...] = jnp.maximum(
        agg @ wr_ref[...] + x_ref[...] @ wx_ref[...] + b_ref[...], 0.0)

  return pl.pallas_call(
      body,
      grid=(n // bn,),
      in_specs=[
          pl.BlockSpec((bn, fi), lambda i: (i, 0)),
          pl.BlockSpec((_NC, bn, fi), lambda i: (0, i, 0)),
          pl.BlockSpec((fi, fo), lambda i: (0, 0)),
          pl.BlockSpec((fi, fo), lambda i: (0, 0)),
          pl.BlockSpec((1, fo), lambda i: (0, 0)),
      ],
      out_specs=pl.BlockSpec((bn, fo), lambda i: (i, 0)),
      out_shape=jax.ShapeDtypeStruct((n, fo), jnp.float32),
  )(x, aggp, w_rel, w_root, b)


def _avgpool_finish(poolp, cntp):
  n = poolp.shape[1]
  bn = 2000

  def body(p_ref, c_ref, o_ref):
    ssum = p_ref[0] + p_ref[1]
    cnt = jnp.maximum(c_ref[0][:, 0:1] + c_ref[1][:, 0:1], 1.0)
    o_ref[...] = ssum / cnt

  return pl.pallas_call(
      body,
      grid=(n // bn,),
      in_specs=[
          pl.BlockSpec((_NC, bn, 64), lambda i: (0, i, 0)),
          pl.BlockSpec((_NC, bn, 16), lambda i: (0, i, 0)),
      ],
      out_specs=pl.BlockSpec((bn, 64), lambda i: (i, 0)),
      out_shape=jax.ShapeDtypeStruct((n, 64), jnp.float32),
  )(poolp, cntp)


def _graph_dense_c4(xa, iso, aggp, isoaggp, wr_a, wr_b, wx_a, wx_b, b):
  n = xa.shape[0]
  fo = wr_a.shape[1]
  bn = 2000

  def body(x_ref, i_ref, a_ref, ia_ref, wra_ref, wrb_ref, wxa_ref, wxb_ref,
           b_ref, o_ref):
    agg = a_ref[0] + a_ref[1]
    iagg = ia_ref[0] + ia_ref[1]
    o_ref[...] = jnp.maximum(
        agg @ wra_ref[...] + iagg @ wrb_ref[...]
        + x_ref[...] @ wxa_ref[...] + i_ref[...] @ wxb_ref[...]
        + b_ref[...], 0.0)

  return pl.pallas_call(
      body,
      grid=(n // bn,),
      in_specs=[
          pl.BlockSpec((bn, 64), lambda i: (i, 0)),
          pl.BlockSpec((bn, _NUM_I2), lambda i: (i, 0)),
          pl.BlockSpec((_NC, bn, 64), lambda i: (0, i, 0)),
          pl.BlockSpec((_NC, bn, _NUM_I2), lambda i: (0, i, 0)),
          pl.BlockSpec((64, fo), lambda i: (0, 0)),
          pl.BlockSpec((_NUM_I2, fo), lambda i: (0, 0)),
          pl.BlockSpec((64, fo), lambda i: (0, 0)),
          pl.BlockSpec((_NUM_I2, fo), lambda i: (0, 0)),
          pl.BlockSpec((1, fo), lambda i: (0, 0)),
      ],
      out_specs=pl.BlockSpec((bn, fo), lambda i: (i, 0)),
      out_shape=jax.ShapeDtypeStruct((n, fo), jnp.float32),
  )(xa, iso, aggp, isoaggp, wr_a, wr_b, wx_a, wx_b, b)


def _head(x1gp, x2gp, u0, u1, w0, b0, w1, b1, wl, bl, pl0, pl1):
  eps = 1e-7
  temp = 0.1

  def body(x1_ref, x2_ref, u0_ref, u1_ref, w0_ref, b0_ref, w1_ref, b1_ref,
           wl_ref, bl_ref, p0_ref, p1_ref, o_ref, reg_ref):
    z = jnp.concatenate(
        [x1_ref[0] + x1_ref[1], x2_ref[0] + x2_ref[1]], axis=1)

    def drop(zin, u, p):
      dp = (jnp.log(p + eps) - jnp.log(1.0 - p + eps)
            + jnp.log(u + eps) - jnp.log(1.0 - u + eps))
      dp = 1.0 / (1.0 + jnp.exp(-dp / temp))
      return zin * (1.0 - dp) / (1.0 - p)

    def regterm(w, b, p, d):
      sum_sq = jnp.sum(w * w) + jnp.sum(b * b)
      ent = p * jnp.log(p + eps) + (1.0 - p) * jnp.log(1.0 - p + eps)
      return _W_REG * sum_sq / (1.0 - p) + _D_REG * d * ent

    p0 = 1.0 / (1.0 + jnp.exp(-p0_ref[0, 0]))
    p1 = 1.0 / (1.0 + jnp.exp(-p1_ref[0, 0]))
    z = jnp.maximum(drop(z, u0_ref[...], p0) @ w0_ref[...] + b0_ref[...], 0.0)
    z = jnp.maximum(drop(z, u1_ref[...], p1) @ w1_ref[...] + b1_ref[...], 0.0)
    o_ref[...] = z @ wl_ref[...] + bl_ref[...]
    reg = (regterm(w0_ref[...], b0_ref[...], p0, 2.0 * _DIM)
           + regterm(w1_ref[...], b1_ref[...], p1, 1.0 * _DIM))
    reg_ref[...] = jnp.broadcast_to(reg, (1, 1))

  return pl.pallas_call(
      body,
      out_shape=(
          jax.ShapeDtypeStruct((_G, 2), jnp.float32),
          jax.ShapeDtypeStruct((1, 1), jnp.float32),
      ),
  )(x1gp, x2gp, u0, u1, w0, b0, w1, b1, wl, bl, pl0, pl1)


def _expand_reduce_mats(fi, fo, fo_pad):
  k = fi * fo
  ii = jnp.arange(k, dtype=jnp.int32) // fo
  oo = jnp.arange(k, dtype=jnp.int32) % fo
  r = (jnp.arange(128, dtype=jnp.int32)[:, None] == ii[None, :])
  s = (oo[:, None] == jnp.arange(fo_pad, dtype=jnp.int32)[None, :])
  return r.astype(jnp.float32), s.astype(jnp.float32)


def kernel(x, edge_index, edge_attr, batch, assignment_index_2, iso_type_2,
           edge_index_2, batch_2, params):
  p = params
  f32 = jnp.float32
  row = lambda a: a.reshape(1, -1).astype(f32)

  src = edge_index[0]
  dst = edge_index[1]
  eh = _EPAD // 2
  src_h = _pad_idx(src, _EPAD, 0, 2)
  dst_h = _pad_idx(dst, _EPAD, _N, 2)
  ea_pad = _pad_vals(edge_attr, _EPAD)

  r0, s0 = _expand_reduce_mats(_NUM_FEAT, _DIM // 2, 128)
  r1, s1 = _expand_reduce_mats(_DIM // 2, _DIM, 128)

  a_src = _pad_idx(assignment_index_2[0], _APAD, 0)
  a_dst = _pad_idx(assignment_index_2[1], _APAD, _N2)
  s2_g = _pad_idx(edge_index_2[0], _EPAD, 0)
  d2_s = _pad_idx(edge_index_2[1], _EPAD, _N2)

  def nnconv(xin, w1, b1, w2, b2, r, s_mat, fo, root, bconv):
    aggp = []
    xpad = jnp.concatenate(
        [xin, jnp.zeros((xin.shape[0], 128 - xin.shape[1]), f32)], axis=1)
    for h in range(2):
      xs = _sc_gather(xpad, src_h[h], eh)
      msg = _edge_messages(ea_pad, xs, w1, b1, w2, b2, r, s_mat, 128, h)
      aggp.append(_sc_segsum_lin(msg, dst_h[h], eh, _N))
    return _node_update(xin, root, bconv, aggp[0], aggp[1], fo)

  x1 = nnconv(x, p['nn0_W1'], row(p['nn0_b1']), p['nn0_W2'], row(p['nn0_b2']),
              r0, s0, _DIM // 2, p['conv0_root'], row(p['conv0_b']))
  cntp = _sc_segsum_lin(jnp.ones((_APAD, 16), f32), a_dst, _APAD, _N2)
  isoaggp = _sc_segsum_gather(iso_type_2, s2_g, d2_s, _EPAD, _N2)
  x2 = nnconv(x1, p['nn1_W1'], row(p['nn1_b1']), p['nn1_W2'], row(p['nn1_b2']),
              r1, s1, _DIM, p['conv1_root'], row(p['conv1_b']))

  batch_s = _pad_idx(batch, _NPAD, _G)
  x1gp = _sc_segsum_lin(x2, batch_s, _NPAD, _G)

  poolp = _sc_segsum_gather(x2, a_src, a_dst, _APAD, _N2)
  x2avg = _avgpool_finish(poolp, cntp)

  c4aggp = _sc_segsum_gather(x2avg, s2_g, d2_s, _EPAD, _N2)
  x2b = _graph_dense_c4(x2avg, iso_type_2, c4aggp, isoaggp,
                        p['c4_Wrel'][:64], p['c4_Wrel'][64:],
                        p['c4_Wroot'][:64], p['c4_Wroot'][64:],
                        row(p['c4_b']))
  c5aggp = _sc_segsum_gather(x2b, s2_g, d2_s, _EPAD, _N2)
  x2c = _graph_dense(x2b, c5aggp, p['c5_Wrel'], p['c5_Wroot'],
                     row(p['c5_b']))

  batch2_s = _pad_idx(batch_2, _NPAD, _G)
  x2gp = _sc_segsum_lin(x2c, batch2_s, _NPAD, _G)

  u0 = jax.random.uniform(jax.random.key(101), (_G, 2 * _DIM), dtype=f32)
  u1 = jax.random.uniform(jax.random.key(202), (_G, _DIM), dtype=f32)
  wl = jnp.concatenate([p['mu_W'], p['lv_W']], axis=1)
  bl = jnp.concatenate([p['mu_b'], p['lv_b']]).reshape(1, 2)
  out, reg = _head(x1gp, x2gp, u0, u1,
                   p['fc0_W'], row(p['fc0_b']), p['fc1_W'], row(p['fc1_b']),
                   wl, bl, p['p_logit0'].reshape(1, 1),
                   p['p_logit1'].reshape(1, 1))
  mean = out[:, 0:1]
  log_var = out[:, 1:2]
  return mean, log_var, reg.reshape(())
